# Initial kernel scaffold; baseline (speedup 1.0000x reference)
import jax
import jax.numpy as jnp
from jax.experimental import pallas as pl


def kernel(feature, edge_index, W, b):
    raise NotImplementedError("write your pallas kernel here")


# speedup vs baseline: 6.3517x; 6.3517x over previous
import functools

import jax
import jax.numpy as jnp
from jax import lax
from jax.experimental import pallas as pl
from jax.experimental.pallas import tpu as pltpu
from jax.experimental.pallas import tpu_sc as plsc

N_NODES = 10000
N_EDGES = 320000
D = 128

NC = 2
NS = 16
NW = NC * NS

E_PER_TILE = N_EDGES // NW
CHUNK = 80
N_CHUNKS = E_PER_TILE // CHUNK
N_PAD = 10240
ROWS_PER_TILE = N_PAD // NS

_mesh = plsc.VectorSubcoreMesh(
    core_axis_name="c", subcore_axis_name="s", num_cores=NC, num_subcores=NS
)


@functools.partial(
    pl.kernel,
    out_type=(
        jax.ShapeDtypeStruct((NC, N_PAD, D), jnp.float32),
        jax.ShapeDtypeStruct((NW, N_PAD), jnp.float32),
    ),
    mesh=_mesh,
    compiler_params=pltpu.CompilerParams(needs_layout_passes=False),
    scratch_types=[
        pltpu.VMEM((CHUNK,), jnp.int32),
        pltpu.VMEM((CHUNK,), jnp.int32),
        pltpu.VMEM((CHUNK, D), jnp.float32),
        pltpu.VMEM((N_PAD,), jnp.float32),
        pltpu.VMEM_SHARED((N_PAD, D), jnp.float32),
        pltpu.SemaphoreType.DMA,
    ],
)
def _sc_aggregate(feature, src, dst, zrows, agg_out, deg_out,
                  sidx, didx, rows, deg_v, acc, sem):
    cid = lax.axis_index("c")
    sid = lax.axis_index("s")
    wid = cid * NS + sid

    pltpu.sync_copy(zrows, acc.at[pl.ds(sid * ROWS_PER_TILE, ROWS_PER_TILE)])

    zero16 = jnp.zeros((16,), jnp.float32)

    def _zero_deg(i, carry):
        deg_v[pl.ds(i * 16, 16)] = zero16
        return carry

    lax.fori_loop(0, N_PAD // 16, _zero_deg, 0)
    plsc.subcore_barrier()

    ones16 = jnp.ones((16,), jnp.float32)
    ebase = wid * E_PER_TILE

    def _edge_chunk(c, carry):
        base = ebase + c * CHUNK
        pltpu.sync_copy(src.at[pl.ds(base, CHUNK)], sidx)
        pltpu.sync_copy(dst.at[pl.ds(base, CHUNK)], didx)
        pltpu.async_copy(feature.at[sidx], rows, sem).wait()
        pltpu.sync_copy(rows, acc.at[didx], add=True)
        for j in range(CHUNK // 16):
            idx = didx[pl.ds(j * 16, 16)]
            plsc.addupdate_scatter(deg_v, [idx], ones16)
        return carry

    lax.fori_loop(0, N_CHUNKS, _edge_chunk, 0)
    plsc.subcore_barrier()

    row0 = sid * ROWS_PER_TILE
    pltpu.sync_copy(
        acc.at[pl.ds(row0, ROWS_PER_TILE)],
        agg_out.at[cid, pl.ds(row0, ROWS_PER_TILE)],
    )
    pltpu.sync_copy(deg_v, deg_out.at[wid])


BLK = 2048


def _tc_finish(agg_ref, deg_ref, wt_ref, b_ref, out_ref):
    i = pl.multiple_of(pl.program_id(0) * BLK, 128)
    s = agg_ref[0] + agg_ref[1]
    deg = jnp.sum(deg_ref[:, pl.ds(i, BLK)], axis=0)
    h = s / jnp.maximum(deg, 1.0)[:, None]
    y = jnp.dot(h, wt_ref[...], preferred_element_type=jnp.float32)
    out_ref[...] = jnp.maximum(y + b_ref[...], 0.0)


def kernel(feature, edge_index, W, b):
    src = edge_index[0].astype(jnp.int32)
    dst = edge_index[1].astype(jnp.int32)
    zrows = jnp.zeros((ROWS_PER_TILE, D), jnp.float32)

    agg, degp = _sc_aggregate(feature, src, dst, zrows)

    out = pl.pallas_call(
        _tc_finish,
        grid=(N_PAD // BLK,),
        in_specs=[
            pl.BlockSpec((NC, BLK, D), lambda i: (0, i, 0)),
            pl.BlockSpec((NW, N_PAD), lambda i: (0, 0)),
            pl.BlockSpec((D, D), lambda i: (0, 0)),
            pl.BlockSpec((1, D), lambda i: (0, 0)),
        ],
        out_specs=pl.BlockSpec((BLK, D), lambda i: (i, 0)),
        out_shape=jax.ShapeDtypeStruct((N_PAD, D), jnp.float32),
    )(agg, degp, W.T, b.reshape(1, D))
    return out[:N_NODES]


# speedup vs baseline: 11.2413x; 1.7698x over previous
import functools

import jax
import jax.numpy as jnp
from jax import lax
from jax.experimental import pallas as pl
from jax.experimental.pallas import tpu as pltpu
from jax.experimental.pallas import tpu_sc as plsc

N_NODES = 10000
N_EDGES = 320000
D = 128

NC = 2
NS = 16
NW = NC * NS

E_PER_TILE = N_EDGES // NW
CHUNK = 80
N_CHUNKS = E_PER_TILE // CHUNK
NSTREAM = 4
N_ITERS = N_CHUNKS // NSTREAM
N_PAD = 10240
ROWS_PER_TILE = N_PAD // NS

_mesh = plsc.VectorSubcoreMesh(
    core_axis_name="c", subcore_axis_name="s", num_cores=NC, num_subcores=NS
)


@functools.partial(
    pl.kernel,
    out_type=(
        jax.ShapeDtypeStruct((NC, N_PAD, D), jnp.float32),
        jax.ShapeDtypeStruct((NC, N_PAD), jnp.float32),
    ),
    mesh=_mesh,
    compiler_params=pltpu.CompilerParams(needs_layout_passes=False),
    scratch_types=[
        [pltpu.VMEM((CHUNK,), jnp.int32)] * NSTREAM,
        [pltpu.VMEM((CHUNK,), jnp.int32)] * NSTREAM,
        pltpu.VMEM((NSTREAM, CHUNK, D), jnp.float32),
        pltpu.VMEM((CHUNK,), jnp.float32),
        pltpu.VMEM_SHARED((N_PAD, D), jnp.float32),
        pltpu.VMEM_SHARED((N_PAD,), jnp.float32),
        [pltpu.SemaphoreType.DMA] * NSTREAM,
        [pltpu.SemaphoreType.DMA] * NSTREAM,
        pltpu.SemaphoreType.DMA,
    ],
)
def _sc_aggregate(feature, src, dst, zrows, zdeg, agg_out, deg_out,
                  sbufs, dbufs, rows, ones_v, acc, dacc,
                  isems, gsems, ssem):
    cid = lax.axis_index("c")
    sid = lax.axis_index("s")

    pltpu.sync_copy(zrows, acc.at[pl.ds(sid * ROWS_PER_TILE, ROWS_PER_TILE)])

    @pl.when(sid == 0)
    def _():
        pltpu.sync_copy(zdeg, dacc)

    ones16 = jnp.ones((16,), jnp.float32)
    for k in range(CHUNK // 16):
        ones_v[pl.ds(k * 16, 16)] = ones16

    plsc.subcore_barrier()

    ebase = (cid * NS + sid) * E_PER_TILE

    def _idx_start(c, j):
        base = ebase + c * CHUNK
        pltpu.async_copy(src.at[pl.ds(base, CHUNK)], sbufs[j], isems[j])
        pltpu.async_copy(dst.at[pl.ds(base, CHUNK)], dbufs[j], isems[j])

    def _idx_wait(j):
        pltpu.make_async_copy(src.at[pl.ds(0, CHUNK)], sbufs[j], isems[j]).wait()
        pltpu.make_async_copy(dst.at[pl.ds(0, CHUNK)], dbufs[j], isems[j]).wait()

    def _run_iter(c0, nstream):
        for j in range(nstream):
            _idx_start(c0 + j, j)
        gathers = []
        for j in range(nstream):
            _idx_wait(j)
            gathers.append(
                pltpu.async_copy(feature.at[sbufs[j]], rows.at[j], gsems[j])
            )
        scatters = []
        for j in range(nstream):
            gathers[j].wait()
            scatters.append(
                pltpu.async_copy(rows.at[j], acc.at[dbufs[j]], ssem, add=True)
            )
            scatters.append(
                pltpu.async_copy(ones_v, dacc.at[dbufs[j]], ssem, add=True)
            )
        for s in scatters:
            s.wait()

    def _iter(it, carry):
        _run_iter(it * NSTREAM, NSTREAM)
        return carry

    lax.fori_loop(0, N_ITERS, _iter, 0)
    _run_iter(N_ITERS * NSTREAM, N_CHUNKS - N_ITERS * NSTREAM)

    plsc.subcore_barrier()

    row0 = sid * ROWS_PER_TILE
    pltpu.sync_copy(
        acc.at[pl.ds(row0, ROWS_PER_TILE)],
        agg_out.at[cid, pl.ds(row0, ROWS_PER_TILE)],
    )

    @pl.when(sid == 0)
    def _():
        pltpu.sync_copy(dacc, deg_out.at[cid])


BLK = 2048


def _tc_finish(agg_ref, deg_ref, wt_ref, b_ref, out_ref):
    i = pl.multiple_of(pl.program_id(0) * BLK, 128)
    s = agg_ref[0] + agg_ref[1]
    deg = deg_ref[0, pl.ds(i, BLK)] + deg_ref[1, pl.ds(i, BLK)]
    h = s / jnp.maximum(deg, 1.0)[:, None]
    y = jnp.dot(h, wt_ref[...], preferred_element_type=jnp.float32)
    out_ref[...] = jnp.maximum(y + b_ref[...], 0.0)


def kernel(feature, edge_index, W, b):
    src = edge_index[0].astype(jnp.int32)
    dst = edge_index[1].astype(jnp.int32)
    zrows = jnp.zeros((ROWS_PER_TILE, D), jnp.float32)
    zdeg = jnp.zeros((N_PAD,), jnp.float32)

    agg, degp = _sc_aggregate(feature, src, dst, zrows, zdeg)

    out = pl.pallas_call(
        _tc_finish,
        grid=(N_PAD // BLK,),
        in_specs=[
            pl.BlockSpec((NC, BLK, D), lambda i: (0, i, 0)),
            pl.BlockSpec((NC, N_PAD), lambda i: (0, 0)),
            pl.BlockSpec((D, D), lambda i: (0, 0)),
            pl.BlockSpec((1, D), lambda i: (0, 0)),
        ],
        out_specs=pl.BlockSpec((BLK, D), lambda i: (i, 0)),
        out_shape=jax.ShapeDtypeStruct((N_PAD, D), jnp.float32),
    )(agg, degp, W.T, b.reshape(1, D))
    return out[:N_NODES]


# speedup vs baseline: 11.6574x; 1.0370x over previous
import functools

import jax
import jax.numpy as jnp
from jax import lax
from jax.experimental import pallas as pl
from jax.experimental.pallas import tpu as pltpu
from jax.experimental.pallas import tpu_sc as plsc

N_NODES = 10000
N_EDGES = 320000
D = 128

NC = 2
NS = 16
NW = NC * NS

E_PER_TILE = N_EDGES // NW
CHUNK = 80
N_CHUNKS = E_PER_TILE // CHUNK
NSTREAM = 4
N_ITERS = N_CHUNKS // NSTREAM
N_PAD = 10240
ROWS_PER_TILE = N_PAD // NS

_mesh = plsc.VectorSubcoreMesh(
    core_axis_name="c", subcore_axis_name="s", num_cores=NC, num_subcores=NS
)


@functools.partial(
    pl.kernel,
    out_type=(
        jax.ShapeDtypeStruct((NC, N_PAD, D), jnp.float32),
        jax.ShapeDtypeStruct((NC, N_PAD), jnp.float32),
    ),
    mesh=_mesh,
    compiler_params=pltpu.CompilerParams(needs_layout_passes=False),
    scratch_types=[
        [pltpu.VMEM((CHUNK,), jnp.int32)] * NSTREAM,
        [pltpu.VMEM((CHUNK,), jnp.int32)] * NSTREAM,
        pltpu.VMEM((NSTREAM, CHUNK, D), jnp.float32),
        pltpu.VMEM((CHUNK,), jnp.float32),
        pltpu.VMEM_SHARED((N_PAD, D), jnp.float32),
        pltpu.VMEM_SHARED((N_PAD,), jnp.float32),
        [pltpu.SemaphoreType.DMA] * NSTREAM,
        [pltpu.SemaphoreType.DMA] * NSTREAM,
        pltpu.SemaphoreType.DMA,
    ],
)
def _sc_aggregate(feature, src, dst, zrows, zdeg, agg_out, deg_out,
                  sbufs, dbufs, rows, ones_v, acc, dacc,
                  isems, gsems, ssem):
    cid = lax.axis_index("c")
    sid = lax.axis_index("s")

    pltpu.sync_copy(zrows, acc.at[pl.ds(sid * ROWS_PER_TILE, ROWS_PER_TILE)])

    @pl.when(sid == 0)
    def _():
        pltpu.sync_copy(zdeg, dacc)

    ones16 = jnp.ones((16,), jnp.float32)
    for k in range(CHUNK // 16):
        ones_v[pl.ds(k * 16, 16)] = ones16

    plsc.subcore_barrier()

    ebase = (cid * NS + sid) * E_PER_TILE

    def _idx_start(c, j):
        base = ebase + c * CHUNK
        pltpu.async_copy(src.at[pl.ds(base, CHUNK)], sbufs[j], isems[j])
        pltpu.async_copy(dst.at[pl.ds(base, CHUNK)], dbufs[j], isems[j])

    def _idx_wait(j):
        pltpu.make_async_copy(src.at[pl.ds(0, CHUNK)], sbufs[j], isems[j]).wait()
        pltpu.make_async_copy(dst.at[pl.ds(0, CHUNK)], dbufs[j], isems[j]).wait()

    def _run_iter(c0, nstream):
        for j in range(nstream):
            _idx_start(c0 + j, j)
        gathers = []
        scatters = []
        for j in range(nstream):
            _idx_wait(j)
            gathers.append(
                pltpu.async_copy(feature.at[sbufs[j]], rows.at[j], gsems[j])
            )
            scatters.append(
                pltpu.async_copy(ones_v, dacc.at[dbufs[j]], ssem, add=True)
            )
        for j in range(nstream):
            gathers[j].wait()
            scatters.append(
                pltpu.async_copy(rows.at[j], acc.at[dbufs[j]], ssem, add=True)
            )
        for s in scatters:
            s.wait()

    def _iter(it, carry):
        _run_iter(it * NSTREAM, NSTREAM)
        return carry

    lax.fori_loop(0, N_ITERS, _iter, 0)
    _run_iter(N_ITERS * NSTREAM, N_CHUNKS - N_ITERS * NSTREAM)

    plsc.subcore_barrier()

    row0 = sid * ROWS_PER_TILE
    pltpu.sync_copy(
        acc.at[pl.ds(row0, ROWS_PER_TILE)],
        agg_out.at[cid, pl.ds(row0, ROWS_PER_TILE)],
    )

    @pl.when(sid == 0)
    def _():
        pltpu.sync_copy(dacc, deg_out.at[cid])


BLK = 2048


def _tc_finish(agg_ref, deg_ref, wt_ref, b_ref, out_ref):
    i = pl.multiple_of(pl.program_id(0) * BLK, 128)
    s = agg_ref[0] + agg_ref[1]
    deg = deg_ref[0, pl.ds(i, BLK)] + deg_ref[1, pl.ds(i, BLK)]
    h = s / jnp.maximum(deg, 1.0)[:, None]
    y = jnp.dot(h, wt_ref[...], preferred_element_type=jnp.float32)
    out_ref[...] = jnp.maximum(y + b_ref[...], 0.0)


def kernel(feature, edge_index, W, b):
    src = edge_index[0].astype(jnp.int32)
    dst = edge_index[1].astype(jnp.int32)
    zrows = jnp.zeros((ROWS_PER_TILE, D), jnp.float32)
    zdeg = jnp.zeros((N_PAD,), jnp.float32)

    agg, degp = _sc_aggregate(feature, src, dst, zrows, zdeg)

    out = pl.pallas_call(
        _tc_finish,
        grid=(N_PAD // BLK,),
        in_specs=[
            pl.BlockSpec((NC, BLK, D), lambda i: (0, i, 0)),
            pl.BlockSpec((NC, N_PAD), lambda i: (0, 0)),
            pl.BlockSpec((D, D), lambda i: (0, 0)),
            pl.BlockSpec((1, D), lambda i: (0, 0)),
        ],
        out_specs=pl.BlockSpec((BLK, D), lambda i: (i, 0)),
        out_shape=jax.ShapeDtypeStruct((N_NODES, D), jnp.float32),
    )(agg, degp, W.T, b.reshape(1, D))
    return out


# speedup vs baseline: 11.7827x; 1.0107x over previous
import functools

import jax
import jax.numpy as jnp
from jax import lax
from jax.experimental import pallas as pl
from jax.experimental.pallas import tpu as pltpu
from jax.experimental.pallas import tpu_sc as plsc

N_NODES = 10000
N_EDGES = 320000
D = 128
DH = D // 2

NC = 2
NS = 16
NW = NC * NS

E_PER_TILE = N_EDGES // NS
CHUNK = 80
N_CHUNKS = E_PER_TILE // CHUNK
NSTREAM = 10
N_ITERS = N_CHUNKS // NSTREAM
N_PAD = 10240
ROWS_PER_TILE = N_PAD // NS

_mesh = plsc.VectorSubcoreMesh(
    core_axis_name="c", subcore_axis_name="s", num_cores=NC, num_subcores=NS
)


@functools.partial(
    pl.kernel,
    out_type=(
        jax.ShapeDtypeStruct((NC, N_PAD, DH), jnp.float32),
        jax.ShapeDtypeStruct((1, N_PAD), jnp.float32),
    ),
    mesh=_mesh,
    compiler_params=pltpu.CompilerParams(
        needs_layout_passes=False, use_tc_tiling_on_sc=False
    ),
    scratch_types=[
        [pltpu.VMEM((CHUNK,), jnp.int32)] * NSTREAM,
        [pltpu.VMEM((CHUNK,), jnp.int32)] * NSTREAM,
        pltpu.VMEM((NSTREAM, CHUNK, DH), jnp.float32),
        pltpu.VMEM((CHUNK,), jnp.float32),
        pltpu.VMEM_SHARED((N_PAD, DH), jnp.float32),
        pltpu.VMEM_SHARED((N_PAD,), jnp.float32),
        [pltpu.SemaphoreType.DMA] * NSTREAM,
        [pltpu.SemaphoreType.DMA] * NSTREAM,
        pltpu.SemaphoreType.DMA,
    ],
)
def _sc_aggregate(fhalf, src, dst, zrows, zdeg, agg_out, deg_out,
                  sbufs, dbufs, rows, ones_v, acc, dacc,
                  isems, gsems, ssem):
    cid = lax.axis_index("c")
    sid = lax.axis_index("s")

    pltpu.sync_copy(zrows, acc.at[pl.ds(sid * ROWS_PER_TILE, ROWS_PER_TILE)])

    @pl.when(sid == 0)
    def _():
        pltpu.sync_copy(zdeg, dacc)

    ones16 = jnp.ones((16,), jnp.float32)
    for k in range(CHUNK // 16):
        ones_v[pl.ds(k * 16, 16)] = ones16

    plsc.subcore_barrier()

    ebase = sid * E_PER_TILE
    fh = fhalf.at[cid]
    count_deg = cid == 0

    def _idx_start(c, j):
        base = ebase + c * CHUNK
        pltpu.async_copy(src.at[pl.ds(base, CHUNK)], sbufs[j], isems[j])
        pltpu.async_copy(dst.at[pl.ds(base, CHUNK)], dbufs[j], isems[j])

    def _idx_wait(j):
        pltpu.make_async_copy(src.at[pl.ds(0, CHUNK)], sbufs[j], isems[j]).wait()
        pltpu.make_async_copy(dst.at[pl.ds(0, CHUNK)], dbufs[j], isems[j]).wait()

    def _iter(it, carry):
        c0 = it * NSTREAM
        for j in range(NSTREAM):
            _idx_start(c0 + j, j)
        gathers = []
        scatters = []
        for j in range(NSTREAM):
            _idx_wait(j)
            gathers.append(
                pltpu.async_copy(fh.at[sbufs[j]], rows.at[j], gsems[j])
            )

            @pl.when(count_deg)
            def _(j=j):
                pltpu.async_copy(ones_v, dacc.at[dbufs[j]], ssem, add=True)

        for j in range(NSTREAM):
            gathers[j].wait()
            scatters.append(
                pltpu.async_copy(rows.at[j], acc.at[dbufs[j]], ssem, add=True)
            )
        for s in scatters:
            s.wait()

        @pl.when(count_deg)
        def _():
            for _j in range(NSTREAM):
                pltpu.make_async_copy(ones_v, dacc.at[dbufs[0]], ssem).wait()

        return carry

    lax.fori_loop(0, N_ITERS, _iter, 0)
    plsc.subcore_barrier()

    row0 = sid * ROWS_PER_TILE
    pltpu.sync_copy(
        acc.at[pl.ds(row0, ROWS_PER_TILE)],
        agg_out.at[cid, pl.ds(row0, ROWS_PER_TILE)],
    )

    @pl.when(jnp.logical_and(sid == 0, cid == 0))
    def _():
        pltpu.sync_copy(dacc, deg_out.at[0])


BLK = 2048


def _tc_finish(agg_ref, deg_ref, wt_ref, b_ref, out_ref):
    i = pl.multiple_of(pl.program_id(0) * BLK, 128)
    s = jnp.concatenate([agg_ref[0], agg_ref[1]], axis=1)
    deg = deg_ref[0, pl.ds(i, BLK)]
    h = s / jnp.maximum(deg, 1.0)[:, None]
    y = jnp.dot(h, wt_ref[...], preferred_element_type=jnp.float32)
    out_ref[...] = jnp.maximum(y + b_ref[...], 0.0)


def kernel(feature, edge_index, W, b):
    src = edge_index[0].astype(jnp.int32)
    dst = edge_index[1].astype(jnp.int32)
    fhalf = feature.reshape(N_NODES, NC, DH).transpose(1, 0, 2)
    zrows = jnp.zeros((ROWS_PER_TILE, DH), jnp.float32)
    zdeg = jnp.zeros((N_PAD,), jnp.float32)

    agg, degp = _sc_aggregate(fhalf, src, dst, zrows, zdeg)

    out = pl.pallas_call(
        _tc_finish,
        grid=(N_PAD // BLK,),
        in_specs=[
            pl.BlockSpec((NC, BLK, DH), lambda i: (0, i, 0)),
            pl.BlockSpec((1, N_PAD), lambda i: (0, 0)),
            pl.BlockSpec((D, D), lambda i: (0, 0)),
            pl.BlockSpec((1, D), lambda i: (0, 0)),
        ],
        out_specs=pl.BlockSpec((BLK, D), lambda i: (i, 0)),
        out_shape=jax.ShapeDtypeStruct((N_NODES, D), jnp.float32),
    )(agg, degp, W.T, b.reshape(1, D))
    return out


# speedup vs baseline: 12.1864x; 1.0343x over previous
import functools

import jax
import jax.numpy as jnp
from jax import lax
from jax.experimental import pallas as pl
from jax.experimental.pallas import tpu as pltpu
from jax.experimental.pallas import tpu_sc as plsc

N_NODES = 10000
N_EDGES = 320000
D = 128

NC = 2
NS = 16
NW = NC * NS

E_PER_TILE = N_EDGES // NW
CHUNK = 80
N_CHUNKS = E_PER_TILE // CHUNK
NSTREAM = 4
N_ITERS = N_CHUNKS // NSTREAM
N_PAD = 10240
ROWS_PER_TILE = N_PAD // NS

_mesh = plsc.VectorSubcoreMesh(
    core_axis_name="c", subcore_axis_name="s", num_cores=NC, num_subcores=NS
)


@functools.partial(
    pl.kernel,
    out_type=(
        jax.ShapeDtypeStruct((NC, N_PAD, D), jnp.float32),
        jax.ShapeDtypeStruct((NC, N_PAD), jnp.float32),
    ),
    mesh=_mesh,
    compiler_params=pltpu.CompilerParams(
        needs_layout_passes=False, use_tc_tiling_on_sc=False
    ),
    scratch_types=[
        [pltpu.VMEM((CHUNK,), jnp.int32)] * NSTREAM,
        [pltpu.VMEM((CHUNK,), jnp.int32)] * NSTREAM,
        pltpu.VMEM((NSTREAM, CHUNK, D), jnp.float32),
        pltpu.VMEM((CHUNK,), jnp.float32),
        pltpu.VMEM_SHARED((N_PAD, D), jnp.float32),
        pltpu.VMEM_SHARED((N_PAD,), jnp.float32),
        [pltpu.SemaphoreType.DMA] * NSTREAM,
        [pltpu.SemaphoreType.DMA] * NSTREAM,
        pltpu.SemaphoreType.DMA,
    ],
)
def _sc_aggregate(feature, edges, zrows, zdeg, agg_out, deg_out,
                  sbufs, dbufs, rows, ones_v, acc, dacc,
                  isems, gsems, ssem):
    cid = lax.axis_index("c")
    sid = lax.axis_index("s")

    pltpu.sync_copy(zrows, acc.at[pl.ds(sid * ROWS_PER_TILE, ROWS_PER_TILE)])

    @pl.when(sid == 0)
    def _():
        pltpu.sync_copy(zdeg, dacc)

    ones16 = jnp.ones((16,), jnp.float32)
    for k in range(CHUNK // 16):
        ones_v[pl.ds(k * 16, 16)] = ones16

    plsc.subcore_barrier()

    ebase = (cid * NS + sid) * E_PER_TILE

    def _idx_start(c, j):
        base = ebase + c * CHUNK
        pltpu.async_copy(edges.at[0, pl.ds(base, CHUNK)], sbufs[j], isems[j])
        pltpu.async_copy(edges.at[1, pl.ds(base, CHUNK)], dbufs[j], isems[j])

    def _idx_wait(j):
        pltpu.make_async_copy(edges.at[0, pl.ds(0, CHUNK)], sbufs[j], isems[j]).wait()
        pltpu.make_async_copy(edges.at[1, pl.ds(0, CHUNK)], dbufs[j], isems[j]).wait()

    def _run_iter(c0, nstream):
        for j in range(nstream):
            _idx_start(c0 + j, j)
        gathers = []
        scatters = []
        for j in range(nstream):
            _idx_wait(j)
            gathers.append(
                pltpu.async_copy(feature.at[sbufs[j]], rows.at[j], gsems[j])
            )
            scatters.append(
                pltpu.async_copy(ones_v, dacc.at[dbufs[j]], ssem, add=True)
            )
        for j in range(nstream):
            gathers[j].wait()
            scatters.append(
                pltpu.async_copy(rows.at[j], acc.at[dbufs[j]], ssem, add=True)
            )
        for s in scatters:
            s.wait()

    def _iter(it, carry):
        _run_iter(it * NSTREAM, NSTREAM)
        return carry

    lax.fori_loop(0, N_ITERS, _iter, 0)
    _run_iter(N_ITERS * NSTREAM, N_CHUNKS - N_ITERS * NSTREAM)

    plsc.subcore_barrier()

    row0 = sid * ROWS_PER_TILE
    pltpu.sync_copy(
        acc.at[pl.ds(row0, ROWS_PER_TILE)],
        agg_out.at[cid, pl.ds(row0, ROWS_PER_TILE)],
    )

    @pl.when(sid == 0)
    def _():
        pltpu.sync_copy(dacc, deg_out.at[cid])


BLK = 2048


def _tc_finish(agg_ref, deg_ref, wt_ref, b_ref, out_ref):
    i = pl.multiple_of(pl.program_id(0) * BLK, 128)
    s = agg_ref[0] + agg_ref[1]
    deg = deg_ref[0, pl.ds(i, BLK)] + deg_ref[1, pl.ds(i, BLK)]
    h = s / jnp.maximum(deg, 1.0)[:, None]
    y = jnp.dot(h, wt_ref[...], preferred_element_type=jnp.float32)
    out_ref[...] = jnp.maximum(y + b_ref[...], 0.0)


def kernel(feature, edge_index, W, b):
    edges = edge_index.astype(jnp.int32)
    zrows = jnp.zeros((ROWS_PER_TILE, D), jnp.float32)
    zdeg = jnp.zeros((N_PAD,), jnp.float32)

    agg, degp = _sc_aggregate(feature, edges, zrows, zdeg)

    out = pl.pallas_call(
        _tc_finish,
        grid=(N_PAD // BLK,),
        in_specs=[
            pl.BlockSpec((NC, BLK, D), lambda i: (0, i, 0)),
            pl.BlockSpec((NC, N_PAD), lambda i: (0, 0)),
            pl.BlockSpec((D, D), lambda i: (0, 0)),
            pl.BlockSpec((1, D), lambda i: (0, 0)),
        ],
        out_specs=pl.BlockSpec((BLK, D), lambda i: (i, 0)),
        out_shape=jax.ShapeDtypeStruct((N_NODES, D), jnp.float32),
    )(agg, degp, W.T, b.reshape(1, D))
    return out
